# baseline (speedup 1.0000x reference)
import jax
import jax.numpy as jnp
from jax import lax
from jax.experimental import pallas as pl
from jax.experimental.pallas import tpu as pltpu
from jax.experimental.pallas import tpu_sc as plsc

N_CORES = 2
N_SUBCORES = 16
NW = N_CORES * N_SUBCORES
LANES = 16
BATCH = 16384
K = 32
BPW = BATCH // NW
CHUNK = 128
NCH = BPW // CHUNK


def _mf_body(uid_hbm, iid_hbm, user_hbm, item_hbm, bu_hbm, bi_hbm, bias_hbm,
             out_hbm,
             idx_u, idx_i, urows, irows, bu_v, bi_v, bias_v, out_v,
             sem):
    cid = lax.axis_index("c")
    sid = lax.axis_index("s")
    wid = sid * N_CORES + cid
    base = wid * BPW

    for j in range(NCH):
        pltpu.sync_copy(uid_hbm.at[pl.ds(base + j * CHUNK, CHUNK)], idx_u.at[j])
        pltpu.sync_copy(iid_hbm.at[pl.ds(base + j * CHUNK, CHUNK)], idx_i.at[j])
    pltpu.sync_copy(bias_hbm, bias_v)

    copies = []
    for j in range(NCH):
        copies.append(pltpu.async_copy(
            user_hbm.at[idx_u.at[j]], urows.at[pl.ds(j * CHUNK, CHUNK)], sem))
        copies.append(pltpu.async_copy(
            item_hbm.at[idx_i.at[j]], irows.at[pl.ds(j * CHUNK, CHUNK)], sem))
        copies.append(pltpu.async_copy(
            bu_hbm.at[idx_u.at[j]], bu_v.at[pl.ds(j * CHUNK, CHUNK)], sem))
        copies.append(pltpu.async_copy(
            bi_hbm.at[idx_i.at[j]], bi_v.at[pl.ds(j * CHUNK, CHUNK)], sem))
    for c in copies:
        c.wait()

    bvec = bias_v[...]
    iota16 = lax.iota(jnp.int32, LANES)

    def group_body(g, carry):
        outv = jnp.zeros((LANES,), jnp.float32)
        for l in range(LANES):
            r = g * LANES + l
            u0 = urows[r, pl.ds(0, LANES)]
            u1 = urows[r, pl.ds(LANES, LANES)]
            i0 = irows[r, pl.ds(0, LANES)]
            i1 = irows[r, pl.ds(LANES, LANES)]
            prod = u0 * i0 + u1 * i1
            outv = jnp.where(iota16 == l, jnp.sum(prod), outv)
        bu16 = bu_v[pl.ds(g * LANES, LANES)]
        bi16 = bi_v[pl.ds(g * LANES, LANES)]
        out_v[pl.ds(g * LANES, LANES)] = outv + bu16 + bi16 + bvec
        return carry

    lax.fori_loop(0, BPW // LANES, group_body, 0)

    pltpu.sync_copy(out_v, out_hbm.at[pl.ds(base, BPW)])


_mf = pl.kernel(
    _mf_body,
    mesh=plsc.VectorSubcoreMesh(core_axis_name="c", subcore_axis_name="s"),
    out_type=jax.ShapeDtypeStruct((BATCH,), jnp.float32),
    compiler_params=pltpu.CompilerParams(needs_layout_passes=False,
                                         use_tc_tiling_on_sc=False),
    scratch_types=[
        pltpu.VMEM((NCH, CHUNK), jnp.int32),
        pltpu.VMEM((NCH, CHUNK), jnp.int32),
        pltpu.VMEM((BPW, K), jnp.float32),
        pltpu.VMEM((BPW, K), jnp.float32),
        pltpu.VMEM((BPW,), jnp.float32),
        pltpu.VMEM((BPW,), jnp.float32),
        pltpu.VMEM((LANES,), jnp.float32),
        pltpu.VMEM((BPW,), jnp.float32),
        pltpu.SemaphoreType.DMA,
    ],
)


def kernel(train_x, user_w, item_w, bias_user_w, bias_item_w, bias):
    uid = train_x[:, 0]
    iid = train_x[:, 1]
    bias16 = jnp.broadcast_to(bias, (LANES,))
    return _mf(uid, iid, user_w, item_w,
               bias_user_w.reshape(-1), bias_item_w.reshape(-1), bias16)
